# baseline (speedup 1.0000x reference)
import jax
import jax.numpy as jnp
from jax import lax
from jax.experimental import pallas as pl
from jax.experimental.pallas import tpu as pltpu
from jax.experimental.pallas import tpu_sc as plsc

_NC = 2
_NS = 16
_L = 16
_NW = _NC * _NS

_B, _K, _D = 16384, 32, 64
_BPW = _B // _NW
_GCHUNK = 128


def _sc_body(pi_hbm, g_hbm, mu_hbm, sg_hbm,
             mu_out, sg_out, idx_out,
             pi_v, g_v, idx_v, gidx_v, mu_v, sg_v, sem_mu, sem_sg):
  wid = lax.axis_index("s") * _NC + lax.axis_index("c")
  base = wid * _BPW

  pltpu.sync_copy(pi_hbm.at[wid], pi_v)
  pltpu.sync_copy(g_hbm.at[wid], g_v)

  lanes = lax.iota(jnp.int32, _L)

  def argmax_group(j, _):
    off = j * _L
    sl = pl.ds(off, _L)
    best_v = pi_v[0, sl] + g_v[0, sl]
    best_i = jnp.zeros((_L,), jnp.int32)
    for k in range(1, _K):
      v = pi_v[k, sl] + g_v[k, sl]
      p = v > best_v
      best_v = jnp.where(p, v, best_v)
      best_i = jnp.where(p, jnp.int32(k), best_i)
    idx_v[sl] = best_i
    gidx_v[sl] = (base + off + lanes) * _K + best_i
    return 0

  lax.fori_loop(0, _BPW // _L, argmax_group, 0, unroll=False)

  copies = []
  for c in range(_BPW // _GCHUNK):
    isl = pl.ds(c * _GCHUNK, _GCHUNK)
    copies.append(pltpu.async_copy(mu_hbm.at[gidx_v.at[isl]],
                                   mu_v.at[isl], sem_mu))
    copies.append(pltpu.async_copy(sg_hbm.at[gidx_v.at[isl]],
                                   sg_v.at[isl], sem_sg))
  pltpu.sync_copy(idx_v, idx_out.at[pl.ds(base, _BPW)])
  for cp in copies:
    cp.wait()

  pltpu.sync_copy(mu_v, mu_out.at[pl.ds(base, _BPW)])
  pltpu.sync_copy(sg_v, sg_out.at[pl.ds(base, _BPW)])


@jax.jit
def kernel(pi, mu, sigma):
  B, K = pi.shape
  D = mu.shape[2]
  g = jax.random.gumbel(jax.random.key(42), (B, K), pi.dtype)

  pi_b = pi.T.reshape(K, _NW, _BPW).transpose(1, 0, 2)
  g_b = g.T.reshape(K, _NW, _BPW).transpose(1, 0, 2)
  mu_flat = mu.reshape(B * K, D)
  sg_flat = sigma.reshape(B * K, D)

  mesh = plsc.VectorSubcoreMesh(core_axis_name="c", subcore_axis_name="s")
  run = pl.kernel(
      _sc_body,
      out_type=(
          jax.ShapeDtypeStruct((B, D), mu.dtype),
          jax.ShapeDtypeStruct((B, D), sigma.dtype),
          jax.ShapeDtypeStruct((B,), jnp.int32),
      ),
      mesh=mesh,
      compiler_params=pltpu.CompilerParams(use_tc_tiling_on_sc=False),
      scratch_types=[
          pltpu.VMEM((K, _BPW), jnp.float32),
          pltpu.VMEM((K, _BPW), jnp.float32),
          pltpu.VMEM((_BPW,), jnp.int32),
          pltpu.VMEM((_BPW,), jnp.int32),
          pltpu.VMEM((_BPW, _D), jnp.float32),
          pltpu.VMEM((_BPW, _D), jnp.float32),
          pltpu.SemaphoreType.DMA,
          pltpu.SemaphoreType.DMA,
      ],
  )
  mu_sel, sg_sel, idx = run(pi_b, g_b, mu_flat, sg_flat)
  return mu_sel, sg_sel, idx


# speedup vs baseline: 4.2459x; 4.2459x over previous
import jax
import jax.numpy as jnp
from jax import lax
from jax.experimental import pallas as pl
from jax.experimental.pallas import tpu as pltpu
from jax.experimental.pallas import tpu_sc as plsc

_NC = 2
_NS = 16
_L = 16
_NW = _NC * _NS

_B, _K, _D = 16384, 32, 64
_BPW = _B // _NW
_NBT = _BPW // 128
_NDT = _D // 8
_M = _NDT * (_B // 128)


def _sc_body(pi_hbm, g_hbm, mu_hbm, sg_hbm,
             omu, osg, idx_out,
             pi_v, g_v, idx_v, slab0, slab1, tile_v, sem0, sem1):
  wid = lax.axis_index("s") * _NC + lax.axis_index("c")
  base = wid * _BPW
  bt0 = wid * _NBT

  def slab_src(ref, it):
    bt = bt0 + it // _NDT
    dt = it % _NDT
    return ref.at[:, dt * (_B // 128) + bt]

  slabs = (slab0, slab1)
  sems = (sem0, sem1)

  pltpu.async_copy(slab_src(mu_hbm, 0), slab0, sem0)

  pltpu.sync_copy(pi_hbm.at[:, pl.ds(base, _BPW)], pi_v)
  pltpu.sync_copy(g_hbm.at[:, pl.ds(base, _BPW)], g_v)

  def argmax_group(j, _):
    sl = pl.ds(j * _L, _L)
    best_v = pi_v[0, sl] + g_v[0, sl]
    best_i = jnp.zeros((_L,), jnp.int32)
    for k in range(1, _K):
      v = pi_v[k, sl] + g_v[k, sl]
      p = v > best_v
      best_v = jnp.where(p, v, best_v)
      best_i = jnp.where(p, jnp.int32(k), best_i)
    idx_v[sl] = best_i
    return 0

  lax.fori_loop(0, _BPW // _L, argmax_group, 0, unroll=False)
  pltpu.sync_copy(idx_v, idx_out.at[pl.ds(base, _BPW)])

  lanes = lax.iota(jnp.int32, _L)
  nslab = _NBT * _NDT

  def run_table(src_hbm, dst_hbm, nxt_hbm):
    def two_slabs(t, _):
      for j in range(2):
        it = t * 2 + j
        buf = slabs[j]
        sem = sems[j]
        pltpu.make_async_copy(slab_src(src_hbm, it), buf, sem).wait()

        lo = (it // _NDT) * 128
        for lg in range(8):
          k16 = idx_v[pl.ds(lo + lg * _L, _L)]
          l16 = lanes + lg * _L
          for ds_ in range(8):
            d16 = jnp.full((_L,), ds_, jnp.int32)
            tile_v[ds_, pl.ds(lg * _L, _L)] = plsc.load_gather(
                buf, [k16, d16, l16])

        bt = bt0 + it // _NDT
        dt = it % _NDT
        pltpu.sync_copy(
            tile_v, dst_hbm.at[pl.ds(dt * 8, 8), pl.ds(bt * 128, 128)])

        nit = it + 2
        @pl.when(nit < nslab)
        def _():
          pltpu.async_copy(slab_src(src_hbm, nit), buf, sem)
        if nxt_hbm is not None:
          @pl.when(nit == nslab)
          def _():
            pltpu.async_copy(slab_src(nxt_hbm, 0), buf, sem)
          @pl.when(nit == nslab + 1)
          def _():
            pltpu.async_copy(slab_src(nxt_hbm, 1), buf, sem)
      return 0

    lax.fori_loop(0, nslab // 2, two_slabs, 0, unroll=False)

  pltpu.async_copy(slab_src(mu_hbm, 1), slab1, sem1)
  run_table(mu_hbm, omu, sg_hbm)
  run_table(sg_hbm, osg, None)


@jax.jit
def kernel(pi, mu, sigma):
  B, K = pi.shape
  D = mu.shape[2]
  g = jax.random.gumbel(jax.random.key(42), (B, K), pi.dtype)

  piT = pi.T
  gT = g.T
  mu5 = mu.transpose(1, 2, 0).reshape(K, _NDT, 8, B // 128, 128)
  mu5 = mu5.transpose(0, 1, 3, 2, 4).reshape(K, _M, 8, 128)
  sg5 = sigma.transpose(1, 2, 0).reshape(K, _NDT, 8, B // 128, 128)
  sg5 = sg5.transpose(0, 1, 3, 2, 4).reshape(K, _M, 8, 128)

  mesh = plsc.VectorSubcoreMesh(core_axis_name="c", subcore_axis_name="s")
  run = pl.kernel(
      _sc_body,
      out_type=(
          jax.ShapeDtypeStruct((D, B), mu.dtype),
          jax.ShapeDtypeStruct((D, B), sigma.dtype),
          jax.ShapeDtypeStruct((B,), jnp.int32),
      ),
      mesh=mesh,
      compiler_params=pltpu.CompilerParams(needs_layout_passes=False),
      scratch_types=[
          pltpu.VMEM((K, _BPW), jnp.float32),
          pltpu.VMEM((K, _BPW), jnp.float32),
          pltpu.VMEM((_BPW,), jnp.int32),
          pltpu.VMEM((K, 8, 128), jnp.float32),
          pltpu.VMEM((K, 8, 128), jnp.float32),
          pltpu.VMEM((8, 128), jnp.float32),
          pltpu.SemaphoreType.DMA,
          pltpu.SemaphoreType.DMA,
      ],
  )
  omu, osg, idx = run(piT, gT, mu5, sg5)
  return omu.T, osg.T, idx
